# Initial kernel scaffold; baseline (speedup 1.0000x reference)
import functools

import jax
import jax.numpy as jnp
from jax import lax
from jax.experimental import pallas as pl
from jax.experimental.pallas import tpu as pltpu
from jax.experimental.pallas import tpu_sc as plsc

NC = 2
NS = 16
NW = NC * NS
CHUNK = 128


def _sc_degree(np_, nch):
    rpt = np_ // NS
    mesh = plsc.VectorSubcoreMesh(core_axis_name="c", subcore_axis_name="s")

    @functools.partial(
        pl.kernel,
        out_type=jax.ShapeDtypeStruct((NC, np_), jnp.float32),
        mesh=mesh,
        scratch_types=[
            pltpu.VMEM((nch, CHUNK), jnp.int32),
            pltpu.VMEM((CHUNK,), jnp.float32),
            pltpu.VMEM_SHARED((np_,), jnp.float32),
        ],
    )
    def k(col_hbm, ones_hbm, out_hbm, col_v, ones_v, acc):
        c = lax.axis_index("c")
        s = lax.axis_index("s")
        w = c * NS + s
        pltpu.sync_copy(col_hbm.at[w], col_v)
        pltpu.sync_copy(ones_hbm.at[pl.ds(0, CHUNK)], ones_v)
        base = s * rpt
        pltpu.sync_copy(ones_hbm.at[pl.ds(base, rpt)], acc.at[pl.ds(base, rpt)])
        plsc.subcore_barrier()

        @pl.loop(0, nch)
        def _(j):
            pltpu.sync_copy(ones_v, acc.at[col_v.at[j]], add=True)

        plsc.subcore_barrier()
        pltpu.sync_copy(acc.at[pl.ds(base, rpt)], out_hbm.at[c, pl.ds(base, rpt)])

    return k


def _sc_scatter(np_, d, nch):
    rpt = np_ // NS
    mesh = plsc.VectorSubcoreMesh(core_axis_name="c", subcore_axis_name="s")

    @functools.partial(
        pl.kernel,
        out_type=jax.ShapeDtypeStruct((NC, np_, d), jnp.float32),
        mesh=mesh,
        scratch_types=[
            pltpu.VMEM((nch, CHUNK), jnp.int32),
            pltpu.VMEM((nch, CHUNK), jnp.int32),
            pltpu.VMEM((CHUNK, d), jnp.float32),
            pltpu.VMEM((CHUNK, d), jnp.float32),
            pltpu.VMEM_SHARED((np_, d), jnp.float32),
            pltpu.SemaphoreType.DMA,
            pltpu.SemaphoreType.DMA,
        ],
    )
    def k(g_hbm, row_hbm, col_hbm, out_hbm,
          row_v, col_v, buf0, buf1, acc, sem0, sem1):
        c = lax.axis_index("c")
        s = lax.axis_index("s")
        w = c * NS + s
        pltpu.sync_copy(row_hbm.at[w], row_v)
        pltpu.sync_copy(col_hbm.at[w], col_v)
        base = s * rpt
        pltpu.sync_copy(g_hbm.at[pl.ds(base, rpt)], acc.at[pl.ds(base, rpt)])
        plsc.subcore_barrier()

        pltpu.async_copy(g_hbm.at[row_v.at[0]], buf0, sem0)

        def phase(j, buf_cur, sem_cur, buf_nxt, sem_nxt):
            pltpu.make_async_copy(g_hbm.at[row_v.at[j]], buf_cur, sem_cur).wait()

            @pl.when(j + 1 < nch)
            def _():
                pltpu.async_copy(g_hbm.at[row_v.at[j + 1]], buf_nxt, sem_nxt)

            pltpu.sync_copy(buf_cur, acc.at[col_v.at[j]], add=True)

        @pl.loop(0, nch // 2)
        def _(i):
            j0 = i * 2
            phase(j0, buf0, sem0, buf1, sem1)
            phase(j0 + 1, buf1, sem1, buf0, sem0)

        plsc.subcore_barrier()
        pltpu.sync_copy(acc.at[pl.ds(base, rpt)],
                        out_hbm.at[c, pl.ds(base, rpt)])

    return k


def _tc_first(np_, din, dh, n, blk):
    def body(x_ref, w_ref, deg_ref, g_ref, dinv_ref):
        i = pl.program_id(0)
        deg = deg_ref[0] + deg_ref[1] - 1.0
        dinv = lax.rsqrt(deg)
        rows = i * blk + lax.broadcasted_iota(jnp.int32, (blk, 1), 0)
        h = lax.dot_general(x_ref[...], w_ref[...],
                            (((1,), (1,)), ((), ())),
                            preferred_element_type=jnp.float32)
        g_ref[...] = jnp.where(rows < n, h * dinv[:, None], 0.0)
        dinv_ref[...] = dinv

    return pl.pallas_call(
        body,
        grid=(np_ // blk,),
        in_specs=[
            pl.BlockSpec((blk, din), lambda i: (i, 0)),
            pl.BlockSpec((dh, din), lambda i: (0, 0)),
            pl.BlockSpec((NC, blk), lambda i: (0, i)),
        ],
        out_specs=[
            pl.BlockSpec((blk, dh), lambda i: (i, 0)),
            pl.BlockSpec((blk,), lambda i: (i,)),
        ],
        out_shape=[
            jax.ShapeDtypeStruct((np_, dh), jnp.float32),
            jax.ShapeDtypeStruct((np_,), jnp.float32),
        ],
    )


def _tc_mid(np_, dh, n, blk):
    def body(p_ref, g_ref, dinv_ref, b_ref, w_ref, gout_ref):
        i = pl.program_id(0)
        dinv = dinv_ref[...]
        pre = (p_ref[0] + p_ref[1] - g_ref[...]) * dinv[:, None] + b_ref[...][None, :]
        h = jnp.where(pre >= 0, pre, 0.01 * pre)
        rows = i * blk + lax.broadcasted_iota(jnp.int32, (blk, 1), 0)
        hw = lax.dot_general(h, w_ref[...], (((1,), (1,)), ((), ())),
                             preferred_element_type=jnp.float32)
        gout_ref[...] = jnp.where(rows < n, hw * dinv[:, None], 0.0)

    return pl.pallas_call(
        body,
        grid=(np_ // blk,),
        in_specs=[
            pl.BlockSpec((NC, blk, dh), lambda i: (0, i, 0)),
            pl.BlockSpec((blk, dh), lambda i: (i, 0)),
            pl.BlockSpec((blk,), lambda i: (i,)),
            pl.BlockSpec((dh,), lambda i: (0,)),
            pl.BlockSpec((dh, dh), lambda i: (0, 0)),
        ],
        out_specs=pl.BlockSpec((blk, dh), lambda i: (i, 0)),
        out_shape=jax.ShapeDtypeStruct((np_, dh), jnp.float32),
    )


def _tc_epilogue(np_, dh, blk):
    def body(p_ref, g_ref, dinv_ref, b_ref, out_ref):
        dinv = dinv_ref[...]
        pre = (p_ref[0] + p_ref[1] - g_ref[...]) * dinv[:, None] + b_ref[...][None, :]
        out_ref[...] = jnp.where(pre >= 0, pre, 0.01 * pre)

    return pl.pallas_call(
        body,
        grid=(np_ // blk,),
        in_specs=[
            pl.BlockSpec((NC, blk, dh), lambda i: (0, i, 0)),
            pl.BlockSpec((blk, dh), lambda i: (i, 0)),
            pl.BlockSpec((blk,), lambda i: (i,)),
            pl.BlockSpec((dh,), lambda i: (0,)),
        ],
        out_specs=pl.BlockSpec((blk, dh), lambda i: (i, 0)),
        out_shape=jax.ShapeDtypeStruct((np_, dh), jnp.float32),
    )


@jax.jit
def _run(x, edge_index, W1, b1, W2, b2, W3, b3):
    n, din = x.shape
    dh = W1.shape[0]
    e = edge_index.shape[1]
    blk = 512

    np_ = -(-n // blk) * blk
    nch = -(-e // (NW * CHUNK))
    nch += nch % 2
    e_pad = NW * nch * CHUNK

    row = jnp.concatenate(
        [edge_index[0], jnp.full((e_pad - e,), n, jnp.int32)]).reshape(NW, nch, CHUNK)
    col = jnp.concatenate(
        [edge_index[1], jnp.full((e_pad - e,), np_ - 1, jnp.int32)]).reshape(NW, nch, CHUNK)
    xp = jnp.pad(x, ((0, np_ - n), (0, 0)))
    ones = jnp.ones((np_,), jnp.float32)

    deg_p = _sc_degree(np_, nch)(col, ones)
    g1, dinv = _tc_first(np_, din, dh, n, blk)(xp, W1, deg_p)

    scat = _sc_scatter(np_, dh, nch)
    mid = _tc_mid(np_, dh, n, blk)

    p1 = scat(g1, row, col)
    g2 = mid(p1, g1, dinv, b1, W2)
    p2 = scat(g2, row, col)
    g3 = mid(p2, g2, dinv, b2, W3)
    p3 = scat(g3, row, col)
    h = _tc_epilogue(np_, dh, blk)(p3, g3, dinv, b3)
    return h[:n]


def kernel(x, edge_index, batch, W1, b1, W2, b2, W3, b3):
    del batch
    return _run(x, edge_index, W1, b1, W2, b2, W3, b3)


# speedup vs baseline: 11.4201x; 11.4201x over previous
import functools

import jax
import jax.numpy as jnp
from jax import lax
from jax.experimental import pallas as pl
from jax.experimental.pallas import tpu as pltpu
from jax.experimental.pallas import tpu_sc as plsc

NC = 2
NS = 16
NW = NC * NS
CHUNK = 128


def _sc_degree(np_, nch):
    rpt = np_ // NS
    mesh = plsc.VectorSubcoreMesh(core_axis_name="c", subcore_axis_name="s")

    @functools.partial(
        pl.kernel,
        out_type=jax.ShapeDtypeStruct((NC, np_), jnp.float32),
        mesh=mesh,
        compiler_params=pltpu.CompilerParams(use_tc_tiling_on_sc=False),
        scratch_types=[
            pltpu.VMEM((nch, CHUNK), jnp.int32),
            pltpu.VMEM((CHUNK,), jnp.float32),
            pltpu.VMEM_SHARED((np_,), jnp.float32),
        ],
    )
    def k(col_hbm, ones_hbm, out_hbm, col_v, ones_v, acc):
        c = lax.axis_index("c")
        s = lax.axis_index("s")
        w = c * NS + s
        pltpu.sync_copy(col_hbm.at[w], col_v)
        pltpu.sync_copy(ones_hbm.at[pl.ds(0, CHUNK)], ones_v)
        base = s * rpt
        pltpu.sync_copy(ones_hbm.at[pl.ds(base, rpt)], acc.at[pl.ds(base, rpt)])
        plsc.subcore_barrier()

        @pl.loop(0, nch)
        def _(j):
            pltpu.sync_copy(ones_v, acc.at[col_v.at[j]], add=True)

        plsc.subcore_barrier()
        pltpu.sync_copy(acc.at[pl.ds(base, rpt)], out_hbm.at[c, pl.ds(base, rpt)])

    return k


def _sc_scatter(np_, d, nch):
    rpt = np_ // NS
    mesh = plsc.VectorSubcoreMesh(core_axis_name="c", subcore_axis_name="s")

    @functools.partial(
        pl.kernel,
        out_type=jax.ShapeDtypeStruct((NC, np_, d), jnp.float32),
        mesh=mesh,
        compiler_params=pltpu.CompilerParams(use_tc_tiling_on_sc=False),
        scratch_types=[
            pltpu.VMEM((nch, CHUNK), jnp.int32),
            pltpu.VMEM((nch, CHUNK), jnp.int32),
            pltpu.VMEM((CHUNK, d), jnp.float32),
            pltpu.VMEM((CHUNK, d), jnp.float32),
            pltpu.VMEM_SHARED((np_, d), jnp.float32),
            pltpu.SemaphoreType.DMA,
            pltpu.SemaphoreType.DMA,
        ],
    )
    def k(g_hbm, row_hbm, col_hbm, out_hbm,
          row_v, col_v, buf0, buf1, acc, sem0, sem1):
        c = lax.axis_index("c")
        s = lax.axis_index("s")
        w = c * NS + s
        pltpu.sync_copy(row_hbm.at[w], row_v)
        pltpu.sync_copy(col_hbm.at[w], col_v)
        base = s * rpt
        pltpu.sync_copy(g_hbm.at[pl.ds(base, rpt)], acc.at[pl.ds(base, rpt)])
        plsc.subcore_barrier()

        pltpu.async_copy(g_hbm.at[row_v.at[0]], buf0, sem0)

        def phase(j, buf_cur, sem_cur, buf_nxt, sem_nxt):
            pltpu.make_async_copy(g_hbm.at[row_v.at[j]], buf_cur, sem_cur).wait()

            @pl.when(j + 1 < nch)
            def _():
                pltpu.async_copy(g_hbm.at[row_v.at[j + 1]], buf_nxt, sem_nxt)

            pltpu.sync_copy(buf_cur, acc.at[col_v.at[j]], add=True)

        @pl.loop(0, nch // 2)
        def _(i):
            j0 = i * 2
            phase(j0, buf0, sem0, buf1, sem1)
            phase(j0 + 1, buf1, sem1, buf0, sem0)

        plsc.subcore_barrier()
        pltpu.sync_copy(acc.at[pl.ds(base, rpt)],
                        out_hbm.at[c, pl.ds(base, rpt)])

    return k


def _tc_first(np_, din, dh, n, blk):
    def body(x_ref, w_ref, deg_ref, g_ref, dinv_ref):
        i = pl.program_id(0)
        deg = deg_ref[0] + deg_ref[1] - 1.0
        dinv = lax.rsqrt(deg)
        rows = i * blk + lax.broadcasted_iota(jnp.int32, (blk, 1), 0)
        h = lax.dot_general(x_ref[...], w_ref[...],
                            (((1,), (1,)), ((), ())),
                            preferred_element_type=jnp.float32)
        g_ref[...] = jnp.where(rows < n, h * dinv[:, None], 0.0)
        dinv_ref[...] = dinv

    return pl.pallas_call(
        body,
        grid=(np_ // blk,),
        in_specs=[
            pl.BlockSpec((blk, din), lambda i: (i, 0)),
            pl.BlockSpec((dh, din), lambda i: (0, 0)),
            pl.BlockSpec((NC, blk), lambda i: (0, i)),
        ],
        out_specs=[
            pl.BlockSpec((blk, dh), lambda i: (i, 0)),
            pl.BlockSpec((blk,), lambda i: (i,)),
        ],
        out_shape=[
            jax.ShapeDtypeStruct((np_, dh), jnp.float32),
            jax.ShapeDtypeStruct((np_,), jnp.float32),
        ],
    )


def _tc_mid(np_, dh, n, blk):
    def body(p_ref, g_ref, dinv_ref, b_ref, w_ref, gout_ref):
        i = pl.program_id(0)
        dinv = dinv_ref[...]
        pre = (p_ref[0] + p_ref[1] - g_ref[...]) * dinv[:, None] + b_ref[...][None, :]
        h = jnp.where(pre >= 0, pre, 0.01 * pre)
        rows = i * blk + lax.broadcasted_iota(jnp.int32, (blk, 1), 0)
        hw = lax.dot_general(h, w_ref[...], (((1,), (1,)), ((), ())),
                             preferred_element_type=jnp.float32)
        gout_ref[...] = jnp.where(rows < n, hw * dinv[:, None], 0.0)

    return pl.pallas_call(
        body,
        grid=(np_ // blk,),
        in_specs=[
            pl.BlockSpec((NC, blk, dh), lambda i: (0, i, 0)),
            pl.BlockSpec((blk, dh), lambda i: (i, 0)),
            pl.BlockSpec((blk,), lambda i: (i,)),
            pl.BlockSpec((dh,), lambda i: (0,)),
            pl.BlockSpec((dh, dh), lambda i: (0, 0)),
        ],
        out_specs=pl.BlockSpec((blk, dh), lambda i: (i, 0)),
        out_shape=jax.ShapeDtypeStruct((np_, dh), jnp.float32),
    )


def _tc_epilogue(np_, dh, blk):
    def body(p_ref, g_ref, dinv_ref, b_ref, out_ref):
        dinv = dinv_ref[...]
        pre = (p_ref[0] + p_ref[1] - g_ref[...]) * dinv[:, None] + b_ref[...][None, :]
        out_ref[...] = jnp.where(pre >= 0, pre, 0.01 * pre)

    return pl.pallas_call(
        body,
        grid=(np_ // blk,),
        in_specs=[
            pl.BlockSpec((NC, blk, dh), lambda i: (0, i, 0)),
            pl.BlockSpec((blk, dh), lambda i: (i, 0)),
            pl.BlockSpec((blk,), lambda i: (i,)),
            pl.BlockSpec((dh,), lambda i: (0,)),
        ],
        out_specs=pl.BlockSpec((blk, dh), lambda i: (i, 0)),
        out_shape=jax.ShapeDtypeStruct((np_, dh), jnp.float32),
    )


@jax.jit
def _run(x, edge_index, W1, b1, W2, b2, W3, b3):
    n, din = x.shape
    dh = W1.shape[0]
    e = edge_index.shape[1]
    blk = 512

    np_ = -(-n // blk) * blk
    nch = -(-e // (NW * CHUNK))
    nch += nch % 2
    e_pad = NW * nch * CHUNK

    row = jnp.concatenate(
        [edge_index[0], jnp.full((e_pad - e,), n, jnp.int32)]).reshape(NW, nch, CHUNK)
    col = jnp.concatenate(
        [edge_index[1], jnp.full((e_pad - e,), np_ - 1, jnp.int32)]).reshape(NW, nch, CHUNK)
    xp = jnp.pad(x, ((0, np_ - n), (0, 0)))
    ones = jnp.ones((np_,), jnp.float32)

    deg_p = _sc_degree(np_, nch)(col, ones)
    g1, dinv = _tc_first(np_, din, dh, n, blk)(xp, W1, deg_p)

    scat = _sc_scatter(np_, dh, nch)
    mid = _tc_mid(np_, dh, n, blk)

    p1 = scat(g1, row, col)
    g2 = mid(p1, g1, dinv, b1, W2)
    p2 = scat(g2, row, col)
    g3 = mid(p2, g2, dinv, b2, W3)
    p3 = scat(g3, row, col)
    h = _tc_epilogue(np_, dh, blk)(p3, g3, dinv, b3)
    return h[:n]


def kernel(x, edge_index, batch, W1, b1, W2, b2, W3, b3):
    del batch
    return _run(x, edge_index, W1, b1, W2, b2, W3, b3)


# speedup vs baseline: 12.2889x; 1.0761x over previous
import functools

import jax
import jax.numpy as jnp
from jax import lax
from jax.experimental import pallas as pl
from jax.experimental.pallas import tpu as pltpu
from jax.experimental.pallas import tpu_sc as plsc

NC = 2
NS = 16
NW = NC * NS
CHUNK = 128


def _sc_degree(np_, nch):
    rpt = np_ // NS
    mesh = plsc.VectorSubcoreMesh(core_axis_name="c", subcore_axis_name="s")

    @functools.partial(
        pl.kernel,
        out_type=jax.ShapeDtypeStruct((NC, np_), jnp.float32),
        mesh=mesh,
        compiler_params=pltpu.CompilerParams(use_tc_tiling_on_sc=False),
        scratch_types=[
            pltpu.VMEM((nch, CHUNK), jnp.int32),
            pltpu.VMEM((CHUNK,), jnp.float32),
            pltpu.VMEM_SHARED((np_,), jnp.float32),
        ],
    )
    def k(col_hbm, ones_hbm, out_hbm, col_v, ones_v, acc):
        c = lax.axis_index("c")
        s = lax.axis_index("s")
        w = c * NS + s
        pltpu.sync_copy(col_hbm.at[w], col_v)
        pltpu.sync_copy(ones_hbm.at[pl.ds(0, CHUNK)], ones_v)
        base = s * rpt
        pltpu.sync_copy(ones_hbm.at[pl.ds(base, rpt)], acc.at[pl.ds(base, rpt)])
        plsc.subcore_barrier()

        @pl.loop(0, nch)
        def _(j):
            pltpu.sync_copy(ones_v, acc.at[col_v.at[j]], add=True)

        plsc.subcore_barrier()
        pltpu.sync_copy(acc.at[pl.ds(base, rpt)], out_hbm.at[c, pl.ds(base, rpt)])

    return k


NBUF = 8
LAG = 4


def _sc_scatter(np_, d, nch):
    rpt = np_ // NS
    mesh = plsc.VectorSubcoreMesh(core_axis_name="c", subcore_axis_name="s")
    assert nch % NBUF == 0

    @functools.partial(
        pl.kernel,
        out_type=jax.ShapeDtypeStruct((NC, np_, d), jnp.float32),
        mesh=mesh,
        compiler_params=pltpu.CompilerParams(use_tc_tiling_on_sc=False),
        scratch_types=(
            [pltpu.VMEM((nch, CHUNK), jnp.int32),
             pltpu.VMEM((nch, CHUNK), jnp.int32)]
            + [pltpu.VMEM((CHUNK, d), jnp.float32) for _ in range(NBUF)]
            + [pltpu.VMEM_SHARED((np_, d), jnp.float32)]
            + [pltpu.SemaphoreType.DMA for _ in range(2 * NBUF)]
        ),
    )
    def k(g_hbm, row_hbm, col_hbm, out_hbm, *scr):
        row_v, col_v = scr[0], scr[1]
        bufs = scr[2:2 + NBUF]
        acc = scr[2 + NBUF]
        gsem = scr[3 + NBUF:3 + 2 * NBUF]
        ssem = scr[3 + 2 * NBUF:3 + 3 * NBUF]

        c = lax.axis_index("c")
        s = lax.axis_index("s")
        w = c * NS + s
        pltpu.sync_copy(row_hbm.at[w], row_v)
        pltpu.sync_copy(col_hbm.at[w], col_v)
        base = s * rpt
        pltpu.sync_copy(g_hbm.at[pl.ds(base, rpt)], acc.at[pl.ds(base, rpt)])
        plsc.subcore_barrier()

        for j0 in range(LAG):
            pltpu.async_copy(g_hbm.at[row_v.at[j0]], bufs[j0], gsem[j0])

        def phase(j, kk):
            b = kk % NBUF
            pltpu.make_async_copy(g_hbm.at[row_v.at[j]], bufs[b], gsem[b]).wait()
            pltpu.async_copy(bufs[b], acc.at[col_v.at[j]], ssem[b], add=True)
            jn = j + LAG
            bn = (kk + LAG) % NBUF

            @pl.when(jn >= NBUF)
            def _():
                pltpu.make_async_copy(
                    bufs[bn], acc.at[col_v.at[jn - NBUF]], ssem[bn]).wait()

            @pl.when(jn < nch)
            def _():
                pltpu.async_copy(g_hbm.at[row_v.at[jn]], bufs[bn], gsem[bn])

        @pl.loop(0, nch // NBUF)
        def _(i):
            for kk in range(NBUF):
                phase(i * NBUF + kk, kk)

        for j in range(nch - LAG, nch):
            b = j % NBUF
            pltpu.make_async_copy(bufs[b], acc.at[col_v.at[j]], ssem[b]).wait()

        plsc.subcore_barrier()
        pltpu.sync_copy(acc.at[pl.ds(base, rpt)],
                        out_hbm.at[c, pl.ds(base, rpt)])

    return k


def _tc_first(np_, din, dh, n, blk):
    def body(x_ref, w_ref, deg_ref, g_ref, dinv_ref):
        i = pl.program_id(0)
        deg = deg_ref[0] + deg_ref[1] - 1.0
        dinv = lax.rsqrt(deg)
        rows = i * blk + lax.broadcasted_iota(jnp.int32, (blk, 1), 0)
        h = lax.dot_general(x_ref[...], w_ref[...],
                            (((1,), (1,)), ((), ())),
                            preferred_element_type=jnp.float32)
        g_ref[...] = jnp.where(rows < n, h * dinv[:, None], 0.0)
        dinv_ref[...] = dinv

    return pl.pallas_call(
        body,
        grid=(np_ // blk,),
        in_specs=[
            pl.BlockSpec((blk, din), lambda i: (i, 0)),
            pl.BlockSpec((dh, din), lambda i: (0, 0)),
            pl.BlockSpec((NC, blk), lambda i: (0, i)),
        ],
        out_specs=[
            pl.BlockSpec((blk, dh), lambda i: (i, 0)),
            pl.BlockSpec((blk,), lambda i: (i,)),
        ],
        out_shape=[
            jax.ShapeDtypeStruct((np_, dh), jnp.float32),
            jax.ShapeDtypeStruct((np_,), jnp.float32),
        ],
    )


def _tc_mid(np_, dh, n, blk):
    def body(p_ref, g_ref, dinv_ref, b_ref, w_ref, gout_ref):
        i = pl.program_id(0)
        dinv = dinv_ref[...]
        pre = (p_ref[0] + p_ref[1] - g_ref[...]) * dinv[:, None] + b_ref[...][None, :]
        h = jnp.where(pre >= 0, pre, 0.01 * pre)
        rows = i * blk + lax.broadcasted_iota(jnp.int32, (blk, 1), 0)
        hw = lax.dot_general(h, w_ref[...], (((1,), (1,)), ((), ())),
                             preferred_element_type=jnp.float32)
        gout_ref[...] = jnp.where(rows < n, hw * dinv[:, None], 0.0)

    return pl.pallas_call(
        body,
        grid=(np_ // blk,),
        in_specs=[
            pl.BlockSpec((NC, blk, dh), lambda i: (0, i, 0)),
            pl.BlockSpec((blk, dh), lambda i: (i, 0)),
            pl.BlockSpec((blk,), lambda i: (i,)),
            pl.BlockSpec((dh,), lambda i: (0,)),
            pl.BlockSpec((dh, dh), lambda i: (0, 0)),
        ],
        out_specs=pl.BlockSpec((blk, dh), lambda i: (i, 0)),
        out_shape=jax.ShapeDtypeStruct((np_, dh), jnp.float32),
    )


def _tc_epilogue(np_, dh, blk):
    def body(p_ref, g_ref, dinv_ref, b_ref, out_ref):
        dinv = dinv_ref[...]
        pre = (p_ref[0] + p_ref[1] - g_ref[...]) * dinv[:, None] + b_ref[...][None, :]
        out_ref[...] = jnp.where(pre >= 0, pre, 0.01 * pre)

    return pl.pallas_call(
        body,
        grid=(np_ // blk,),
        in_specs=[
            pl.BlockSpec((NC, blk, dh), lambda i: (0, i, 0)),
            pl.BlockSpec((blk, dh), lambda i: (i, 0)),
            pl.BlockSpec((blk,), lambda i: (i,)),
            pl.BlockSpec((dh,), lambda i: (0,)),
        ],
        out_specs=pl.BlockSpec((blk, dh), lambda i: (i, 0)),
        out_shape=jax.ShapeDtypeStruct((np_, dh), jnp.float32),
    )


@jax.jit
def _run(x, edge_index, W1, b1, W2, b2, W3, b3):
    n, din = x.shape
    dh = W1.shape[0]
    e = edge_index.shape[1]
    blk = 512

    np_ = -(-n // blk) * blk
    nch = -(-e // (NW * CHUNK))
    nch = -(-nch // NBUF) * NBUF
    e_pad = NW * nch * CHUNK

    row = jnp.concatenate(
        [edge_index[0], jnp.full((e_pad - e,), n, jnp.int32)]).reshape(NW, nch, CHUNK)
    col = jnp.concatenate(
        [edge_index[1], jnp.full((e_pad - e,), np_ - 1, jnp.int32)]).reshape(NW, nch, CHUNK)
    xp = jnp.pad(x, ((0, np_ - n), (0, 0)))
    ones = jnp.ones((np_,), jnp.float32)

    deg_p = _sc_degree(np_, nch)(col, ones)
    g1, dinv = _tc_first(np_, din, dh, n, blk)(xp, W1, deg_p)

    scat = _sc_scatter(np_, dh, nch)
    mid = _tc_mid(np_, dh, n, blk)

    p1 = scat(g1, row, col)
    g2 = mid(p1, g1, dinv, b1, W2)
    p2 = scat(g2, row, col)
    g3 = mid(p2, g2, dinv, b2, W3)
    p3 = scat(g3, row, col)
    h = _tc_epilogue(np_, dh, blk)(p3, g3, dinv, b3)
    return h[:n]


def kernel(x, edge_index, batch, W1, b1, W2, b2, W3, b3):
    del batch
    return _run(x, edge_index, W1, b1, W2, b2, W3, b3)


# speedup vs baseline: 32.1225x; 2.6139x over previous
import functools

import jax
import jax.numpy as jnp
from jax import lax
from jax.experimental import pallas as pl
from jax.experimental.pallas import tpu as pltpu
from jax.experimental.pallas import tpu_sc as plsc

NC = 2
NS = 16
NW = NC * NS
CHUNK = 128


def _sc_degree(np_, nch):
    rpt = np_ // NS
    mesh = plsc.VectorSubcoreMesh(core_axis_name="c", subcore_axis_name="s")

    @functools.partial(
        pl.kernel,
        out_type=jax.ShapeDtypeStruct((NC, np_), jnp.float32),
        mesh=mesh,
        compiler_params=pltpu.CompilerParams(use_tc_tiling_on_sc=False),
        scratch_types=[
            pltpu.VMEM((nch, CHUNK), jnp.int32),
            pltpu.VMEM((CHUNK,), jnp.float32),
            pltpu.VMEM_SHARED((np_,), jnp.float32),
        ],
    )
    def k(col_hbm, ones_hbm, out_hbm, col_v, ones_v, acc):
        c = lax.axis_index("c")
        s = lax.axis_index("s")
        w = c * NS + s
        pltpu.sync_copy(col_hbm.at[w], col_v)
        pltpu.sync_copy(ones_hbm.at[pl.ds(0, CHUNK)], ones_v)
        base = s * rpt
        pltpu.sync_copy(ones_hbm.at[pl.ds(base, rpt)], acc.at[pl.ds(base, rpt)])
        plsc.subcore_barrier()

        @pl.loop(0, nch)
        def _(j):
            pltpu.sync_copy(ones_v, acc.at[col_v.at[j]], add=True)

        plsc.subcore_barrier()
        pltpu.sync_copy(acc.at[pl.ds(base, rpt)], out_hbm.at[c, pl.ds(base, rpt)])

    return k


NBUF = 8
LAG = 4


def _sc_scatter(np_, d, dhalf, nch):
    rpt = np_ // NS
    mesh = plsc.VectorSubcoreMesh(core_axis_name="c", subcore_axis_name="s")
    assert nch % NBUF == 0

    @functools.partial(
        pl.kernel,
        out_type=jax.ShapeDtypeStruct((NC, np_, dhalf), jnp.float32),
        mesh=mesh,
        compiler_params=pltpu.CompilerParams(use_tc_tiling_on_sc=False),
        scratch_types=(
            [pltpu.VMEM((nch, CHUNK), jnp.int32),
             pltpu.VMEM((nch, CHUNK), jnp.int32)]
            + [pltpu.VMEM((CHUNK, dhalf), jnp.float32) for _ in range(NBUF)]
            + [pltpu.VMEM_SHARED((np_, dhalf), jnp.float32),
               pltpu.VMEM_SHARED((np_, dhalf), jnp.float32)]
            + [pltpu.SemaphoreType.DMA for _ in range(2 * NBUF)]
        ),
    )
    def k(g_hbm, row_hbm, col_hbm, out_hbm, *scr):
        row_v, col_v = scr[0], scr[1]
        bufs = scr[2:2 + NBUF]
        acc = scr[2 + NBUF]
        g_sp = scr[3 + NBUF]
        gsem = scr[4 + NBUF:4 + 2 * NBUF]
        ssem = scr[4 + 2 * NBUF:4 + 3 * NBUF]

        c = lax.axis_index("c")
        s = lax.axis_index("s")
        pltpu.sync_copy(row_hbm.at[s], row_v)
        pltpu.sync_copy(col_hbm.at[s], col_v)
        base = s * rpt
        fbase = c * dhalf
        pltpu.sync_copy(g_hbm.at[pl.ds(base, rpt), pl.ds(fbase, dhalf)],
                        g_sp.at[pl.ds(base, rpt)])
        pltpu.sync_copy(g_hbm.at[pl.ds(base, rpt), pl.ds(fbase, dhalf)],
                        acc.at[pl.ds(base, rpt)])
        plsc.subcore_barrier()

        for j0 in range(LAG):
            pltpu.async_copy(g_sp.at[row_v.at[j0]], bufs[j0], gsem[j0])

        def phase(j, kk):
            b = kk % NBUF
            pltpu.make_async_copy(g_sp.at[row_v.at[j]], bufs[b], gsem[b]).wait()
            pltpu.async_copy(bufs[b], acc.at[col_v.at[j]], ssem[b], add=True)
            jn = j + LAG
            bn = (kk + LAG) % NBUF

            @pl.when(jn >= NBUF)
            def _():
                pltpu.make_async_copy(
                    bufs[bn], acc.at[col_v.at[jn - NBUF]], ssem[bn]).wait()

            @pl.when(jn < nch)
            def _():
                pltpu.async_copy(g_sp.at[row_v.at[jn]], bufs[bn], gsem[bn])

        @pl.loop(0, nch // NBUF)
        def _(i):
            for kk in range(NBUF):
                phase(i * NBUF + kk, kk)

        for j in range(nch - LAG, nch):
            b = j % NBUF
            pltpu.make_async_copy(bufs[b], acc.at[col_v.at[j]], ssem[b]).wait()

        plsc.subcore_barrier()
        pltpu.sync_copy(acc.at[pl.ds(base, rpt)],
                        out_hbm.at[c, pl.ds(base, rpt)])

    return k


def _tc_first(np_, din, dh, n, blk):
    def body(x_ref, w_ref, deg_ref, g_ref, dinv_ref):
        i = pl.program_id(0)
        deg = deg_ref[0] + deg_ref[1] - 1.0
        dinv = lax.rsqrt(deg)
        rows = i * blk + lax.broadcasted_iota(jnp.int32, (blk, 1), 0)
        h = lax.dot_general(x_ref[...], w_ref[...],
                            (((1,), (1,)), ((), ())),
                            preferred_element_type=jnp.float32)
        g_ref[...] = jnp.where(rows < n, h * dinv[:, None], 0.0)
        dinv_ref[...] = dinv

    return pl.pallas_call(
        body,
        grid=(np_ // blk,),
        in_specs=[
            pl.BlockSpec((blk, din), lambda i: (i, 0)),
            pl.BlockSpec((dh, din), lambda i: (0, 0)),
            pl.BlockSpec((NC, blk), lambda i: (0, i)),
        ],
        out_specs=[
            pl.BlockSpec((blk, dh), lambda i: (i, 0)),
            pl.BlockSpec((blk,), lambda i: (i,)),
        ],
        out_shape=[
            jax.ShapeDtypeStruct((np_, dh), jnp.float32),
            jax.ShapeDtypeStruct((np_,), jnp.float32),
        ],
    )


def _tc_mid(np_, dh, dhalf, n, blk):
    def body(p_ref, dinv_ref, b_ref, w_ref, gout_ref):
        i = pl.program_id(0)
        dinv = dinv_ref[...]
        agg = jnp.concatenate([p_ref[0], p_ref[1]], axis=1)
        pre = agg * dinv[:, None] + b_ref[...][None, :]
        h = jnp.where(pre >= 0, pre, 0.01 * pre)
        rows = i * blk + lax.broadcasted_iota(jnp.int32, (blk, 1), 0)
        hw = lax.dot_general(h, w_ref[...], (((1,), (1,)), ((), ())),
                             preferred_element_type=jnp.float32)
        gout_ref[...] = jnp.where(rows < n, hw * dinv[:, None], 0.0)

    return pl.pallas_call(
        body,
        grid=(np_ // blk,),
        in_specs=[
            pl.BlockSpec((NC, blk, dhalf), lambda i: (0, i, 0)),
            pl.BlockSpec((blk,), lambda i: (i,)),
            pl.BlockSpec((dh,), lambda i: (0,)),
            pl.BlockSpec((dh, dh), lambda i: (0, 0)),
        ],
        out_specs=pl.BlockSpec((blk, dh), lambda i: (i, 0)),
        out_shape=jax.ShapeDtypeStruct((np_, dh), jnp.float32),
    )


def _tc_epilogue(np_, dh, dhalf, blk):
    def body(p_ref, dinv_ref, b_ref, out_ref):
        dinv = dinv_ref[...]
        agg = jnp.concatenate([p_ref[0], p_ref[1]], axis=1)
        pre = agg * dinv[:, None] + b_ref[...][None, :]
        out_ref[...] = jnp.where(pre >= 0, pre, 0.01 * pre)

    return pl.pallas_call(
        body,
        grid=(np_ // blk,),
        in_specs=[
            pl.BlockSpec((NC, blk, dhalf), lambda i: (0, i, 0)),
            pl.BlockSpec((blk,), lambda i: (i,)),
            pl.BlockSpec((dh,), lambda i: (0,)),
        ],
        out_specs=pl.BlockSpec((blk, dh), lambda i: (i, 0)),
        out_shape=jax.ShapeDtypeStruct((np_, dh), jnp.float32),
    )


@jax.jit
def _run(x, edge_index, W1, b1, W2, b2, W3, b3):
    n, din = x.shape
    dh = W1.shape[0]
    e = edge_index.shape[1]
    blk = 512

    np_ = -(-n // blk) * blk
    nch = -(-e // (NS * CHUNK))
    nch = -(-nch // (2 * NBUF)) * (2 * NBUF)
    e_pad = NS * nch * CHUNK
    nchd = nch // 2

    row = jnp.concatenate(
        [edge_index[0], jnp.full((e_pad - e,), n, jnp.int32)])
    col = jnp.concatenate(
        [edge_index[1], jnp.full((e_pad - e,), np_ - 1, jnp.int32)])
    row_s = row.reshape(NS, nch, CHUNK)
    col_s = col.reshape(NS, nch, CHUNK)
    col_d = col.reshape(NW, nchd, CHUNK)
    xp = jnp.pad(x, ((0, np_ - n), (0, 0)))
    ones = jnp.ones((np_,), jnp.float32)

    deg_p = _sc_degree(np_, nchd)(col_d, ones)
    g1, dinv = _tc_first(np_, din, dh, n, blk)(xp, W1, deg_p)

    dhalf = dh // NC
    scat = _sc_scatter(np_, dh, dhalf, nch)
    mid = _tc_mid(np_, dh, dhalf, n, blk)

    p1 = scat(g1, row_s, col_s)
    g2 = mid(p1, dinv, b1, W2)
    p2 = scat(g2, row_s, col_s)
    g3 = mid(p2, dinv, b2, W3)
    p3 = scat(g3, row_s, col_s)
    h = _tc_epilogue(np_, dh, dhalf, blk)(p3, dinv, b3)
    return h[:n]


def kernel(x, edge_index, batch, W1, b1, W2, b2, W3, b3):
    del batch
    return _run(x, edge_index, W1, b1, W2, b2, W3, b3)


# speedup vs baseline: 32.1748x; 1.0016x over previous
import functools

import jax
import jax.numpy as jnp
from jax import lax
from jax.experimental import pallas as pl
from jax.experimental.pallas import tpu as pltpu
from jax.experimental.pallas import tpu_sc as plsc

NC = 2
NS = 16
NW = NC * NS
CHUNK = 128


def _sc_degree(np_, nch):
    rpt = np_ // NS
    mesh = plsc.VectorSubcoreMesh(core_axis_name="c", subcore_axis_name="s")

    @functools.partial(
        pl.kernel,
        out_type=jax.ShapeDtypeStruct((NC, np_), jnp.float32),
        mesh=mesh,
        compiler_params=pltpu.CompilerParams(use_tc_tiling_on_sc=False),
        scratch_types=[
            pltpu.VMEM((nch, CHUNK), jnp.int32),
            pltpu.VMEM((CHUNK,), jnp.float32),
            pltpu.VMEM_SHARED((np_,), jnp.float32),
        ],
    )
    def k(col_hbm, ones_hbm, out_hbm, col_v, ones_v, acc):
        c = lax.axis_index("c")
        s = lax.axis_index("s")
        w = c * NS + s
        pltpu.sync_copy(col_hbm.at[w], col_v)
        pltpu.sync_copy(ones_hbm.at[pl.ds(0, CHUNK)], ones_v)
        base = s * rpt
        pltpu.sync_copy(ones_hbm.at[pl.ds(base, rpt)], acc.at[pl.ds(base, rpt)])
        plsc.subcore_barrier()

        @pl.loop(0, nch)
        def _(j):
            pltpu.sync_copy(ones_v, acc.at[col_v.at[j]], add=True)

        plsc.subcore_barrier()
        pltpu.sync_copy(acc.at[pl.ds(base, rpt)], out_hbm.at[c, pl.ds(base, rpt)])

    return k


NBUF = 5
LAG = 2


def _sc_scatter_epi(np_, d, dhalf, nch):
    rpt = np_ // NS
    mesh = plsc.VectorSubcoreMesh(core_axis_name="c", subcore_axis_name="s")
    assert nch % NBUF == 0 and dhalf % 16 == 0

    @functools.partial(
        pl.kernel,
        out_type=jax.ShapeDtypeStruct((NC, np_, dhalf), jnp.float32),
        mesh=mesh,
        compiler_params=pltpu.CompilerParams(use_tc_tiling_on_sc=False),
        scratch_types=(
            [pltpu.VMEM((nch, CHUNK), jnp.int32),
             pltpu.VMEM((nch, CHUNK), jnp.int32)]
            + [pltpu.VMEM((CHUNK, dhalf), jnp.float32) for _ in range(NBUF)]
            + [pltpu.VMEM_SHARED((np_, dhalf), jnp.float32),
               pltpu.VMEM_SHARED((np_, dhalf), jnp.float32)]
            + [pltpu.VMEM((rpt,), jnp.float32),
               pltpu.VMEM((d,), jnp.float32)]
            + [pltpu.SemaphoreType.DMA for _ in range(2 * NBUF)]
        ),
    )
    def k(g_hbm, row_hbm, col_hbm, dinv_hbm, b_hbm, out_hbm, *scr):
        row_v, col_v = scr[0], scr[1]
        bufs = scr[2:2 + NBUF]
        acc = scr[2 + NBUF]
        g_sp = scr[3 + NBUF]
        dinv_v = scr[4 + NBUF]
        b_v = scr[5 + NBUF]
        gsem = scr[6 + NBUF:6 + 2 * NBUF]
        ssem = scr[6 + 2 * NBUF:6 + 3 * NBUF]

        c = lax.axis_index("c")
        s = lax.axis_index("s")
        pltpu.sync_copy(row_hbm.at[s], row_v)
        pltpu.sync_copy(col_hbm.at[s], col_v)
        base = s * rpt
        fbase = c * dhalf
        pltpu.sync_copy(g_hbm.at[pl.ds(base, rpt), pl.ds(fbase, dhalf)],
                        g_sp.at[pl.ds(base, rpt)])
        pltpu.sync_copy(g_hbm.at[pl.ds(base, rpt), pl.ds(fbase, dhalf)],
                        acc.at[pl.ds(base, rpt)])
        pltpu.sync_copy(dinv_hbm.at[pl.ds(base, rpt)], dinv_v)
        pltpu.sync_copy(b_hbm, b_v)
        plsc.subcore_barrier()

        for j0 in range(LAG):
            pltpu.async_copy(g_sp.at[row_v.at[j0]], bufs[j0], gsem[j0])

        def phase(j, kk):
            b = kk % NBUF
            pltpu.make_async_copy(g_sp.at[row_v.at[j]], bufs[b], gsem[b]).wait()
            pltpu.async_copy(bufs[b], acc.at[col_v.at[j]], ssem[b], add=True)
            jn = j + LAG
            bn = (kk + LAG) % NBUF

            @pl.when(jn >= NBUF)
            def _():
                pltpu.make_async_copy(
                    bufs[bn], acc.at[col_v.at[jn - NBUF]], ssem[bn]).wait()

            @pl.when(jn < nch)
            def _():
                pltpu.async_copy(g_sp.at[row_v.at[jn]], bufs[bn], gsem[bn])

        @pl.loop(0, nch // NBUF)
        def _(i):
            for kk in range(NBUF):
                phase(i * NBUF + kk, kk)

        for j in range(nch - (NBUF - LAG), nch):
            b = j % NBUF
            pltpu.make_async_copy(bufs[b], acc.at[col_v.at[j]], ssem[b]).wait()

        plsc.subcore_barrier()

        bparts = [b_v[pl.ds(fbase + 16 * par, 16)] for par in range(dhalf // 16)]

        @pl.loop(0, rpt // CHUNK)
        def _(k2):
            rb = base + CHUNK * k2
            pltpu.sync_copy(acc.at[pl.ds(rb, CHUNK)], bufs[0])

            @pl.loop(0, CHUNK // 16)
            def _(rg):
                dvec = dinv_v[pl.ds(CHUNK * k2 + 16 * rg, 16)]
                for i in range(16):
                    r = 16 * rg + i
                    for par in range(dhalf // 16):
                        v = bufs[0][r, pl.ds(16 * par, 16)]
                        pre = v * dvec[i] + bparts[par]
                        bufs[0][r, pl.ds(16 * par, 16)] = (
                            jnp.where(pre >= 0, pre, 0.01 * pre))

            pltpu.sync_copy(bufs[0], out_hbm.at[c, pl.ds(rb, CHUNK)])

    return k


def _tc_first(np_, din, dh, n, blk):
    def body(x_ref, w_ref, deg_ref, g_ref, dinv_ref):
        i = pl.program_id(0)
        deg = deg_ref[0] + deg_ref[1] - 1.0
        dinv = lax.rsqrt(deg)
        rows = i * blk + lax.broadcasted_iota(jnp.int32, (blk, 1), 0)
        h = lax.dot_general(x_ref[...], w_ref[...],
                            (((1,), (1,)), ((), ())),
                            preferred_element_type=jnp.float32)
        g_ref[...] = jnp.where(rows < n, h * dinv[:, None], 0.0)
        dinv_ref[...] = dinv

    return pl.pallas_call(
        body,
        grid=(np_ // blk,),
        in_specs=[
            pl.BlockSpec((blk, din), lambda i: (i, 0)),
            pl.BlockSpec((dh, din), lambda i: (0, 0)),
            pl.BlockSpec((NC, blk), lambda i: (0, i)),
        ],
        out_specs=[
            pl.BlockSpec((blk, dh), lambda i: (i, 0)),
            pl.BlockSpec((blk,), lambda i: (i,)),
        ],
        out_shape=[
            jax.ShapeDtypeStruct((np_, dh), jnp.float32),
            jax.ShapeDtypeStruct((np_,), jnp.float32),
        ],
    )


def _tc_mid(np_, dh, dhalf, n, blk):
    def body(h_ref, dinv_ref, w_ref, gout_ref):
        i = pl.program_id(0)
        dinv = dinv_ref[...]
        h = jnp.concatenate([h_ref[0], h_ref[1]], axis=1)
        rows = i * blk + lax.broadcasted_iota(jnp.int32, (blk, 1), 0)
        hw = lax.dot_general(h, w_ref[...], (((1,), (1,)), ((), ())),
                             preferred_element_type=jnp.float32)
        gout_ref[...] = jnp.where(rows < n, hw * dinv[:, None], 0.0)

    return pl.pallas_call(
        body,
        grid=(np_ // blk,),
        in_specs=[
            pl.BlockSpec((NC, blk, dhalf), lambda i: (0, i, 0)),
            pl.BlockSpec((blk,), lambda i: (i,)),
            pl.BlockSpec((dh, dh), lambda i: (0, 0)),
        ],
        out_specs=pl.BlockSpec((blk, dh), lambda i: (i, 0)),
        out_shape=jax.ShapeDtypeStruct((np_, dh), jnp.float32),
    )


@jax.jit
def _run(x, edge_index, W1, b1, W2, b2, W3, b3):
    n, din = x.shape
    dh = W1.shape[0]
    e = edge_index.shape[1]
    blk = 512

    np_ = -(-n // blk) * blk
    nch = -(-e // (NS * CHUNK))
    nch = -(-nch // (2 * NBUF)) * (2 * NBUF)
    e_pad = NS * nch * CHUNK
    nchd = nch // 2

    row = jnp.concatenate(
        [edge_index[0], jnp.full((e_pad - e,), n, jnp.int32)])
    col = jnp.concatenate(
        [edge_index[1], jnp.full((e_pad - e,), np_ - 1, jnp.int32)])
    row_s = row.reshape(NS, nch, CHUNK)
    col_s = col.reshape(NS, nch, CHUNK)
    col_d = col.reshape(NW, nchd, CHUNK)
    xp = jnp.pad(x, ((0, np_ - n), (0, 0)))
    ones = jnp.ones((np_,), jnp.float32)

    deg_p = _sc_degree(np_, nchd)(col_d, ones)
    g1, dinv = _tc_first(np_, din, dh, n, blk)(xp, W1, deg_p)

    dhalf = dh // NC
    scat = _sc_scatter_epi(np_, dh, dhalf, nch)
    mid = _tc_mid(np_, dh, dhalf, n, blk)

    h1 = scat(g1, row_s, col_s, dinv, b1)
    g2 = mid(h1, dinv, W2)
    h2 = scat(g2, row_s, col_s, dinv, b2)
    g3 = mid(h2, dinv, W3)
    h = scat(g3, row_s, col_s, dinv, b3)
    return jnp.concatenate([h[0], h[1]], axis=1)[:n]


def kernel(x, edge_index, batch, W1, b1, W2, b2, W3, b3):
    del batch
    return _run(x, edge_index, W1, b1, W2, b2, W3, b3)


# speedup vs baseline: 35.5340x; 1.1044x over previous
import functools

import jax
import jax.numpy as jnp
from jax import lax
from jax.experimental import pallas as pl
from jax.experimental.pallas import tpu as pltpu
from jax.experimental.pallas import tpu_sc as plsc

NC = 2
NS = 16
NW = NC * NS
ROWCHUNK = 128


def _sc_degree(np_, nch, chk):
    rpt = np_ // NS
    mesh = plsc.VectorSubcoreMesh(core_axis_name="c", subcore_axis_name="s")

    @functools.partial(
        pl.kernel,
        out_type=jax.ShapeDtypeStruct((NC, np_), jnp.float32),
        mesh=mesh,
        compiler_params=pltpu.CompilerParams(use_tc_tiling_on_sc=False),
        scratch_types=[
            pltpu.VMEM((nch, chk), jnp.int32),
            pltpu.VMEM((chk,), jnp.float32),
            pltpu.VMEM_SHARED((np_,), jnp.float32),
        ],
    )
    def k(col_hbm, ones_hbm, out_hbm, col_v, ones_v, acc):
        c = lax.axis_index("c")
        s = lax.axis_index("s")
        w = c * NS + s
        pltpu.sync_copy(col_hbm.at[w], col_v)
        pltpu.sync_copy(ones_hbm.at[pl.ds(0, chk)], ones_v)
        base = s * rpt
        pltpu.sync_copy(ones_hbm.at[pl.ds(base, rpt)], acc.at[pl.ds(base, rpt)])
        plsc.subcore_barrier()

        @pl.loop(0, nch)
        def _(j):
            pltpu.sync_copy(ones_v, acc.at[col_v.at[j]], add=True)

        plsc.subcore_barrier()
        pltpu.sync_copy(acc.at[pl.ds(base, rpt)], out_hbm.at[c, pl.ds(base, rpt)])

    return k


NBUF = 5
LAG = 2


def _sc_scatter_epi(np_, d, dhalf, nch, chk):
    rpt = np_ // NS
    mesh = plsc.VectorSubcoreMesh(core_axis_name="c", subcore_axis_name="s")
    assert nch % NBUF == 0 and dhalf % 16 == 0 and chk <= ROWCHUNK

    @functools.partial(
        pl.kernel,
        out_type=jax.ShapeDtypeStruct((np_, d), jnp.float32),
        mesh=mesh,
        compiler_params=pltpu.CompilerParams(use_tc_tiling_on_sc=False),
        scratch_types=(
            [pltpu.VMEM((nch, chk), jnp.int32),
             pltpu.VMEM((nch, chk), jnp.int32)]
            + [pltpu.VMEM((ROWCHUNK, dhalf), jnp.float32) for _ in range(NBUF)]
            + [pltpu.VMEM_SHARED((np_, dhalf), jnp.float32),
               pltpu.VMEM_SHARED((np_, dhalf), jnp.float32)]
            + [pltpu.VMEM((rpt,), jnp.float32),
               pltpu.VMEM((d,), jnp.float32)]
            + [pltpu.SemaphoreType.DMA for _ in range(2 * NBUF)]
        ),
    )
    def k(g_hbm, row_hbm, col_hbm, dinv_hbm, b_hbm, out_hbm, *scr):
        row_v, col_v = scr[0], scr[1]
        bufs = scr[2:2 + NBUF]
        acc = scr[2 + NBUF]
        g_sp = scr[3 + NBUF]
        dinv_v = scr[4 + NBUF]
        b_v = scr[5 + NBUF]
        gsem = scr[6 + NBUF:6 + 2 * NBUF]
        ssem = scr[6 + 2 * NBUF:6 + 3 * NBUF]

        c = lax.axis_index("c")
        s = lax.axis_index("s")
        pltpu.sync_copy(row_hbm.at[s], row_v)
        pltpu.sync_copy(col_hbm.at[s], col_v)
        base = s * rpt
        fbase = c * dhalf
        pltpu.sync_copy(g_hbm.at[pl.ds(base, rpt), pl.ds(fbase, dhalf)],
                        g_sp.at[pl.ds(base, rpt)])
        pltpu.sync_copy(g_hbm.at[pl.ds(base, rpt), pl.ds(fbase, dhalf)],
                        acc.at[pl.ds(base, rpt)])
        pltpu.sync_copy(dinv_hbm.at[pl.ds(base, rpt)], dinv_v)
        pltpu.sync_copy(b_hbm, b_v)
        plsc.subcore_barrier()

        def gbuf(b):
            return bufs[b].at[pl.ds(0, chk)]

        for j0 in range(LAG):
            pltpu.async_copy(g_sp.at[row_v.at[j0]], gbuf(j0), gsem[j0])

        def phase(j, kk):
            b = kk % NBUF
            pltpu.make_async_copy(g_sp.at[row_v.at[j]], gbuf(b), gsem[b]).wait()
            pltpu.async_copy(gbuf(b), acc.at[col_v.at[j]], ssem[b], add=True)
            jn = j + LAG
            bn = (kk + LAG) % NBUF

            @pl.when(jn >= NBUF)
            def _():
                pltpu.make_async_copy(
                    gbuf(bn), acc.at[col_v.at[jn - NBUF]], ssem[bn]).wait()

            @pl.when(jn < nch)
            def _():
                pltpu.async_copy(g_sp.at[row_v.at[jn]], gbuf(bn), gsem[bn])

        @pl.loop(0, nch // NBUF)
        def _(i):
            for kk in range(NBUF):
                phase(i * NBUF + kk, kk)

        for j in range(nch - (NBUF - LAG), nch):
            b = j % NBUF
            pltpu.make_async_copy(gbuf(b), acc.at[col_v.at[j]], ssem[b]).wait()

        plsc.subcore_barrier()

        bparts = [b_v[pl.ds(fbase + 16 * par, 16)] for par in range(dhalf // 16)]

        @pl.loop(0, rpt // ROWCHUNK)
        def _(k2):
            rb = base + ROWCHUNK * k2
            pltpu.sync_copy(acc.at[pl.ds(rb, ROWCHUNK)], bufs[0])

            @pl.loop(0, ROWCHUNK // 16)
            def _(rg):
                dvec = dinv_v[pl.ds(ROWCHUNK * k2 + 16 * rg, 16)]
                for i in range(16):
                    r = 16 * rg + i
                    for par in range(dhalf // 16):
                        v = bufs[0][r, pl.ds(16 * par, 16)]
                        pre = v * dvec[i] + bparts[par]
                        bufs[0][r, pl.ds(16 * par, 16)] = (
                            jnp.where(pre >= 0, pre, 0.01 * pre))

            pltpu.sync_copy(
                bufs[0], out_hbm.at[pl.ds(rb, ROWCHUNK), pl.ds(fbase, dhalf)])

    return k


def _tc_first(np_, din, dh, n, blk):
    def body(x_ref, w_ref, deg_ref, g_ref, dinv_ref):
        i = pl.program_id(0)
        deg = deg_ref[0] + deg_ref[1] - 1.0
        dinv = lax.rsqrt(deg)
        rows = i * blk + lax.broadcasted_iota(jnp.int32, (blk, 1), 0)
        h = lax.dot_general(x_ref[...], w_ref[...],
                            (((1,), (1,)), ((), ())),
                            preferred_element_type=jnp.float32)
        g_ref[...] = jnp.where(rows < n, h * dinv[:, None], 0.0)
        dinv_ref[...] = dinv

    return pl.pallas_call(
        body,
        grid=(np_ // blk,),
        in_specs=[
            pl.BlockSpec((blk, din), lambda i: (i, 0)),
            pl.BlockSpec((dh, din), lambda i: (0, 0)),
            pl.BlockSpec((NC, blk), lambda i: (0, i)),
        ],
        out_specs=[
            pl.BlockSpec((blk, dh), lambda i: (i, 0)),
            pl.BlockSpec((blk,), lambda i: (i,)),
        ],
        out_shape=[
            jax.ShapeDtypeStruct((np_, dh), jnp.float32),
            jax.ShapeDtypeStruct((np_,), jnp.float32),
        ],
    )


def _tc_mid(np_, dh, n, blk):
    def body(h_ref, dinv_ref, w_ref, gout_ref):
        i = pl.program_id(0)
        dinv = dinv_ref[...]
        rows = i * blk + lax.broadcasted_iota(jnp.int32, (blk, 1), 0)
        hw = lax.dot_general(h_ref[...], w_ref[...], (((1,), (1,)), ((), ())),
                             preferred_element_type=jnp.float32)
        gout_ref[...] = jnp.where(rows < n, hw * dinv[:, None], 0.0)

    return pl.pallas_call(
        body,
        grid=(np_ // blk,),
        in_specs=[
            pl.BlockSpec((blk, dh), lambda i: (i, 0)),
            pl.BlockSpec((blk,), lambda i: (i,)),
            pl.BlockSpec((dh, dh), lambda i: (0, 0)),
        ],
        out_specs=pl.BlockSpec((blk, dh), lambda i: (i, 0)),
        out_shape=jax.ShapeDtypeStruct((np_, dh), jnp.float32),
    )


@jax.jit
def _run(x, edge_index, W1, b1, W2, b2, W3, b3):
    n, din = x.shape
    dh = W1.shape[0]
    e = edge_index.shape[1]
    blk = 512

    np_ = -(-n // blk) * blk

    chk = None
    if e % (NS * NC) == 0:
        for cand in range(ROWCHUNK, 63, -1):
            ept = e // NS
            if ept % cand == 0 and (ept // cand) % NBUF == 0:
                chk = cand
                break
    if chk is None:
        chk = ROWCHUNK
        nch = -(-e // (NS * chk))
        nch = -(-nch // (2 * NBUF)) * (2 * NBUF)
        e_pad = NS * nch * chk
        row = jnp.concatenate(
            [edge_index[0], jnp.full((e_pad - e,), n, jnp.int32)])
        col = jnp.concatenate(
            [edge_index[1], jnp.full((e_pad - e,), np_ - 1, jnp.int32)])
    else:
        nch = e // (NS * chk)
        row, col = edge_index[0], edge_index[1]
    nchd = nch // NC
    row_s = row.reshape(NS, nch, chk)
    col_s = col.reshape(NS, nch, chk)
    col_d = col.reshape(NW, nchd, chk)
    xp = jnp.pad(x, ((0, np_ - n), (0, 0)))
    ones = jnp.ones((np_,), jnp.float32)

    deg_p = _sc_degree(np_, nchd, chk)(col_d, ones)
    g1, dinv = _tc_first(np_, din, dh, n, blk)(xp, W1, deg_p)

    dhalf = dh // NC
    scat = _sc_scatter_epi(np_, dh, dhalf, nch, chk)
    mid = _tc_mid(np_, dh, n, blk)

    h1 = scat(g1, row_s, col_s, dinv, b1)
    g2 = mid(h1, dinv, W2)
    h2 = scat(g2, row_s, col_s, dinv, b2)
    g3 = mid(h2, dinv, W3)
    h = scat(g3, row_s, col_s, dinv, b3)
    return h[:n]


def kernel(x, edge_index, batch, W1, b1, W2, b2, W3, b3):
    del batch
    return _run(x, edge_index, W1, b1, W2, b2, W3, b3)
